# Initial kernel scaffold; baseline (speedup 1.0000x reference)
import jax
import jax.numpy as jnp
from jax.experimental import pallas as pl


def kernel(x, w_gate, w_noise, expert_w, expert_b):
    raise NotImplementedError("write your pallas kernel here")


# speedup vs baseline: 1.5453x; 1.5453x over previous
import functools

import jax
import jax.numpy as jnp
from jax.experimental import pallas as pl

N, D, H, E, K = 4096, 1024, 1024, 8, 2
BN = 1024


def _moe_body(gates_ref, x_ref, w_ref, b_ref, o_ref):
    e = pl.program_id(1)
    acc = jnp.dot(x_ref[...], w_ref[0], preferred_element_type=jnp.float32)
    acc = (acc + b_ref[0]) * gates_ref[0]

    @pl.when(e == 0)
    def _():
        o_ref[...] = acc

    @pl.when(e > 0)
    def _():
        o_ref[...] += acc


@functools.partial(jax.jit, static_argnames=())
def _moe_dense(gates, x_bf, w_bf, expert_b):
    grid = (N // BN, E)
    return pl.pallas_call(
        _moe_body,
        grid=grid,
        in_specs=[
            pl.BlockSpec((1, BN, 1), lambda n, e: (e, n, 0)),
            pl.BlockSpec((BN, D), lambda n, e: (n, 0)),
            pl.BlockSpec((1, D, H), lambda n, e: (e, 0, 0)),
            pl.BlockSpec((1, 1, H), lambda n, e: (e, 0, 0)),
        ],
        out_specs=pl.BlockSpec((BN, H), lambda n, e: (n, 0)),
        out_shape=jax.ShapeDtypeStruct((N, H), jnp.float32),
    )(gates, x_bf, w_bf, expert_b)


def kernel(x, w_gate, w_noise, expert_w, expert_b):
    clean_logits = x @ w_gate
    raw_noise_stddev = x @ w_noise
    noise_stddev = jax.nn.softplus(raw_noise_stddev) + 1e-2
    noise = jax.random.normal(jax.random.key(42), clean_logits.shape, dtype=clean_logits.dtype)
    logits = clean_logits + noise * noise_stddev
    top_vals, top_idx = jax.lax.top_k(logits, K)
    top_gates = jax.nn.softmax(top_vals, axis=-1)
    n_tok = x.shape[0]
    gates = jnp.zeros((n_tok, E), dtype=x.dtype).at[
        jnp.arange(n_tok)[:, None], top_idx].set(top_gates)

    gates_t = gates.T[:, :, None]
    x_bf = x.astype(jnp.bfloat16)
    w_bf = expert_w.astype(jnp.bfloat16)
    return _moe_dense(gates_t, x_bf, w_bf, expert_b[:, None, :])


# speedup vs baseline: 1.5467x; 1.0009x over previous
import functools

import jax
import jax.numpy as jnp
from jax.experimental import pallas as pl

N, D, H, E, K = 4096, 1024, 1024, 8, 2
BN = 4096


def _moe_body(gates_ref, x_ref, w_ref, b_ref, o_ref):
    e = pl.program_id(1)
    acc = jnp.dot(x_ref[...], w_ref[0], preferred_element_type=jnp.float32)
    acc = (acc + b_ref[0]) * gates_ref[0]

    @pl.when(e == 0)
    def _():
        o_ref[...] = acc

    @pl.when(e > 0)
    def _():
        o_ref[...] += acc


@functools.partial(jax.jit, static_argnames=())
def _moe_dense(gates, x_bf, w_bf, expert_b):
    grid = (N // BN, E)
    return pl.pallas_call(
        _moe_body,
        grid=grid,
        in_specs=[
            pl.BlockSpec((1, BN, 1), lambda n, e: (e, n, 0)),
            pl.BlockSpec((BN, D), lambda n, e: (n, 0)),
            pl.BlockSpec((1, D, H), lambda n, e: (e, 0, 0)),
            pl.BlockSpec((1, 1, H), lambda n, e: (e, 0, 0)),
        ],
        out_specs=pl.BlockSpec((BN, H), lambda n, e: (n, 0)),
        out_shape=jax.ShapeDtypeStruct((N, H), jnp.float32),
    )(gates, x_bf, w_bf, expert_b)


def kernel(x, w_gate, w_noise, expert_w, expert_b):
    clean_logits = x @ w_gate
    raw_noise_stddev = x @ w_noise
    noise_stddev = jax.nn.softplus(raw_noise_stddev) + 1e-2
    noise = jax.random.normal(jax.random.key(42), clean_logits.shape, dtype=clean_logits.dtype)
    logits = clean_logits + noise * noise_stddev
    top_vals, top_idx = jax.lax.top_k(logits, K)
    top_gates = jax.nn.softmax(top_vals, axis=-1)
    n_tok = x.shape[0]
    gates = jnp.zeros((n_tok, E), dtype=x.dtype).at[
        jnp.arange(n_tok)[:, None], top_idx].set(top_gates)

    gates_t = gates.T[:, :, None]
    x_bf = x.astype(jnp.bfloat16)
    w_bf = expert_w.astype(jnp.bfloat16)
    return _moe_dense(gates_t, x_bf, w_bf, expert_b[:, None, :])
